# Initial kernel scaffold; baseline (speedup 1.0000x reference)
import jax
import jax.numpy as jnp
from jax.experimental import pallas as pl


def kernel(positive_batch, hashes_sorted):
    raise NotImplementedError("write your pallas kernel here")


# speedup vs baseline: 7.4261x; 7.4261x over previous
import functools

import jax
import jax.numpy as jnp
from jax import lax
from jax.experimental import pallas as pl
from jax.experimental.pallas import tpu as pltpu
from jax.experimental.pallas import tpu_sc as plsc

jax.config.update("jax_enable_x64", True)

_NUM_ENTITIES = 1000000
_NUM_NEGS = 16
_ROWW = 32
_NW = 32
_LANES = 16
_SUB = 128
_CHUNK = 512
_CPAD = 0x7FFFFFFF


def _sc_filter_call(total, mp, m):
    qpw = total // _NW
    ppw = qpw // _NUM_NEGS
    nchunk = qpw // _CHUNK
    pchunk = _CHUNK // _NUM_NEGS
    nsub = _CHUNK // _SUB
    half = _NW // 2

    mesh = plsc.VectorSubcoreMesh(core_axis_name="c", subcore_axis_name="s")
    out = jax.ShapeDtypeStruct((total,), jnp.int32)

    @functools.partial(
        pl.kernel,
        out_type=(out, out, out, out),
        mesh=mesh,
        scratch_types=[
            pltpu.VMEM((mp,), jnp.int32),
            pltpu.VMEM((mp,), jnp.int32),
            pltpu.VMEM((ppw,), jnp.int32),
            pltpu.VMEM((ppw,), jnp.int32),
            pltpu.VMEM((ppw,), jnp.int32),
            pltpu.VMEM((qpw,), jnp.int32),
            pltpu.VMEM((_CHUNK,), jnp.int32),
            pltpu.VMEM((_CHUNK,), jnp.int32),
            pltpu.VMEM((_CHUNK // _SUB, _SUB), jnp.int32),
            pltpu.VMEM((_CHUNK, _ROWW), jnp.int32),
            pltpu.VMEM((_CHUNK, _ROWW), jnp.int32),
            pltpu.VMEM((_CHUNK,), jnp.int32),
            pltpu.VMEM((_CHUNK,), jnp.int32),
            pltpu.VMEM((_CHUNK,), jnp.int32),
            pltpu.VMEM((_CHUNK,), jnp.int32),
            pltpu.SemaphoreType.DMA,
        ],
        compiler_params=pltpu.CompilerParams(
            needs_layout_passes=False, use_tc_tiling_on_sc=False),
    )
    def sck(ph_hbm, pr_hbm, pt_hbm, rng_hbm, thi_hbm, tlo_hbm, chi_hbm, clo_hbm,
            oh_hbm, or_hbm, ot_hbm, om_hbm,
            chi_v, clo_v, ph_v, pr_v, pt_v, rng_v, qhi_v, qlo_v, jrow_v,
            whi_v, wlo_v, oh_v, orr_v, ot_v, om_v, sem):
        i32 = jnp.int32
        wid = lax.axis_index("c") * i32(_NW // 2) + lax.axis_index("s")
        pbase = wid * i32(ppw)
        qbase = wid * i32(qpw)
        pltpu.sync_copy(chi_hbm, chi_v)
        pltpu.sync_copy(clo_hbm, clo_v)
        pltpu.sync_copy(ph_hbm.at[pl.ds(pbase, ppw)], ph_v)
        pltpu.sync_copy(pr_hbm.at[pl.ds(pbase, ppw)], pr_v)
        pltpu.sync_copy(pt_hbm.at[pl.ds(pbase, ppw)], pt_v)
        pltpu.sync_copy(rng_hbm.at[pl.ds(qbase, qpw)], rng_v)
        headv = jnp.full((_LANES,), wid, jnp.int32) < i32(half)

        def chunk_body(ci, carry):
            def qgen(p, c):
                pp = ci * i32(pchunk) + p
                pidx = jnp.full((_LANES,), pp, jnp.int32)
                h = plsc.load_gather(ph_v, [pidx])
                r = plsc.load_gather(pr_v, [pidx])
                t = plsc.load_gather(pt_v, [pidx])
                rg = rng_v[pl.ds(pp * i32(_NUM_NEGS), _LANES)]
                orig = jnp.where(headv, h, t)
                corr = rg + ((rg >= orig) & (orig > 0)).astype(jnp.int32)
                nh = jnp.where(headv, corr, h)
                nt = jnp.where(headv, t, corr)
                off = p * i32(_NUM_NEGS)
                qhi_v[pl.ds(off, _LANES)] = nh << 11
                qlo_v[pl.ds(off, _LANES)] = (r << 21) | nt
                oh_v[pl.ds(off, _LANES)] = nh
                orr_v[pl.ds(off, _LANES)] = r
                ot_v[pl.ds(off, _LANES)] = nt
                return c
            lax.fori_loop(jnp.int32(0), jnp.int32(pchunk), qgen, 0)

            for sb in range(nsub):
                def coarse(g, c, sb=sb):
                    off = i32(sb * _SUB) + g * i32(_LANES)
                    q1 = qhi_v[pl.ds(off, _LANES)]
                    q2 = qlo_v[pl.ds(off, _LANES)]
                    j = jnp.zeros((_LANES,), jnp.int32)
                    bit = mp // 2
                    while bit:
                        probe = j + i32(bit - 1)
                        c1 = plsc.load_gather(chi_v, [probe])
                        c2 = plsc.load_gather(clo_v, [probe])
                        lt = (c1 < q1) | ((c1 == q1) & (c2 < q2))
                        j = jnp.where(lt, j + i32(bit), j)
                        bit //= 2
                    jrow_v.at[jnp.int32(sb)][pl.ds(g * i32(_LANES), _LANES)] = j
                    return c
                lax.fori_loop(jnp.int32(0), jnp.int32(_SUB // _LANES), coarse, 0)

            copies = []
            for sb in range(nsub):
                idxr = jrow_v.at[jnp.int32(sb)]
                dst = pl.ds(sb * _SUB, _SUB)
                copies.append(pltpu.async_copy(thi_hbm.at[idxr], whi_v.at[dst], sem))
                copies.append(pltpu.async_copy(tlo_hbm.at[idxr], wlo_v.at[dst], sem))
            for c in copies:
                c.wait()

            def member(g, c):
                off = g * i32(_LANES)
                q1 = qhi_v[pl.ds(off, _LANES)]
                q2 = qlo_v[pl.ds(off, _LANES)]
                rows = jnp.full((_LANES,), off, jnp.int32) + lax.iota(jnp.int32, _LANES)
                acc = jnp.zeros((_LANES,), jnp.bool_)
                for k in range(_ROWW):
                    cols = jnp.full((_LANES,), k, jnp.int32)
                    w1 = plsc.load_gather(whi_v, [rows, cols])
                    w2 = plsc.load_gather(wlo_v, [rows, cols])
                    acc = acc | ((w1 == q1) & (w2 == q2))
                om_v[pl.ds(off, _LANES)] = jnp.where(
                    acc, jnp.zeros((_LANES,), jnp.int32), jnp.ones((_LANES,), jnp.int32))
                return c
            lax.fori_loop(jnp.int32(0), jnp.int32(_CHUNK // _LANES), member, 0)

            obase = qbase + ci * i32(_CHUNK)
            pltpu.sync_copy(oh_v, oh_hbm.at[pl.ds(obase, _CHUNK)])
            pltpu.sync_copy(orr_v, or_hbm.at[pl.ds(obase, _CHUNK)])
            pltpu.sync_copy(ot_v, ot_hbm.at[pl.ds(obase, _CHUNK)])
            pltpu.sync_copy(om_v, om_hbm.at[pl.ds(obase, _CHUNK)])
            return carry

        lax.fori_loop(jnp.int32(0), jnp.int32(nchunk), chunk_body, 0)

    return sck


def kernel(positive_batch, hashes_sorted):
    B = positive_batch.shape[0]
    L = hashes_sorted.shape[0]
    total = B * _NUM_NEGS
    split = total // 2

    key = jax.random.key(12345)
    kh, kt = jax.random.split(key)
    rng_h = jax.random.randint(kh, (split,), 1, _NUM_ENTITIES, dtype=jnp.int64)
    rng_t = jax.random.randint(kt, (total - split,), 1, _NUM_ENTITIES, dtype=jnp.int64)
    rng32 = jnp.concatenate([rng_h, rng_t]).astype(jnp.int32)

    pos_h = positive_batch[:, 0].astype(jnp.int32)
    pos_r = positive_batch[:, 1].astype(jnp.int32)
    pos_t = positive_batch[:, 2].astype(jnp.int32)

    m = L // _ROWW + 1
    lp = m * _ROWW
    pad = (jnp.int64(1) << 62) - 1
    hp = jnp.full((lp,), pad, dtype=jnp.int64).at[:L].set(hashes_sorted)
    thi = (hp >> 31).astype(jnp.int32).reshape(m, _ROWW)
    tlo = (hp & 0x7FFFFFFF).astype(jnp.int32).reshape(m, _ROWW)
    mp = 1 << (m - 1).bit_length()
    chi = jnp.full((mp,), _CPAD, jnp.int32).at[:m].set(thi[:, -1])
    clo = jnp.full((mp,), _CPAD, jnp.int32).at[:m].set(tlo[:, -1])

    out_h, out_r, out_t, out_m = _sc_filter_call(total, mp, m)(
        pos_h, pos_r, pos_t, rng32, thi, tlo, chi, clo)

    neg = jnp.stack([out_h, out_r, out_t], axis=-1).astype(jnp.int64)
    return neg.reshape(B, _NUM_NEGS, 3), (out_m != 0).reshape(B, _NUM_NEGS)
